# Initial kernel scaffold; baseline (speedup 1.0000x reference)
import jax
import jax.numpy as jnp
from jax.experimental import pallas as pl


def kernel(x, qvals, scales, zeros):
    raise NotImplementedError("write your pallas kernel here")


# speedup vs baseline: 1.8944x; 1.8944x over previous
import functools

import jax
import jax.numpy as jnp
from jax import lax
from jax.experimental import pallas as pl
from jax.experimental.pallas import tpu as pltpu
from jax.experimental.pallas import tpu_sc as plsc

NUM_EMB = 1000000
D = 64
DW = D // 4
G = 2
T = 4096 * 50
NW = 32
N_PER = T // NW
C = 128
NCHUNK = N_PER // C


def _body(x_ref, q_ref, s_ref, z_ref, out_ref, idx_v, qv, sv, zv, zf, ov, sem):
    nc = 2
    wid = lax.axis_index("s") * nc + lax.axis_index("c")
    base = wid * N_PER

    lane = lax.iota(jnp.int32, 16)
    halfsel = (lane >= 8).astype(jnp.int32)
    e2 = lane * 2
    cols = [lane * 4 + k for k in range(4)]

    def chunk_body(ci, carry):
        cbase = base + ci * C
        pltpu.sync_copy(x_ref.at[pl.ds(cbase, C)], idx_v)
        cp_q = pltpu.async_copy(q_ref.at[idx_v], qv, sem)
        cp_s = pltpu.async_copy(s_ref.at[idx_v], sv, sem)
        cp_z = pltpu.async_copy(z_ref.at[idx_v], zv, sem)
        cp_q.wait()
        cp_s.wait()
        cp_z.wait()

        def pre_body(j, pcarry):
            zw = zv[pl.ds(j * 16, 16)]
            z0 = ((zw << 24) >> 24).astype(jnp.float32)
            z1 = ((zw << 16) >> 24).astype(jnp.float32)
            pbase = jnp.broadcast_to(j * 32, (16,))
            plsc.store_scatter(zf, [pbase + e2], z0)
            plsc.store_scatter(zf, [pbase + e2 + 1], z1)
            return pcarry

        lax.fori_loop(0, C // 16, pre_body, 0)

        def row_body(i, rcarry):
            w = qv[i, :]
            b0 = (w << 24) >> 24
            b1 = (w << 16) >> 24
            b2 = (w << 8) >> 24
            b3 = w >> 24
            rowv = jnp.broadcast_to(i, (16,))
            idx_sz = jnp.broadcast_to(i * 2, (16,)) + halfsel
            svec = plsc.load_gather(sv, [rowv, halfsel])
            zvec = plsc.load_gather(zf, [idx_sz])
            obase = jnp.broadcast_to(i * 64, (16,))
            for k, bk in enumerate((b0, b1, b2, b3)):
                fk = bk.astype(jnp.float32)
                plsc.store_scatter(ov, [obase + cols[k]], (fk - zvec) * svec)
            return rcarry

        lax.fori_loop(0, C, row_body, 0)
        pltpu.sync_copy(ov, out_ref.at[pl.ds(cbase * D, C * D)])
        return carry

    lax.fori_loop(0, NCHUNK, chunk_body, 0)


_sc_call = functools.partial(
    pl.kernel,
    out_type=jax.ShapeDtypeStruct((T * D,), jnp.float32),
    mesh=plsc.VectorSubcoreMesh(core_axis_name="c", subcore_axis_name="s"),
    compiler_params=pltpu.CompilerParams(
        needs_layout_passes=False, use_tc_tiling_on_sc=False),
    scratch_types=[
        pltpu.VMEM((C,), jnp.int32),
        pltpu.VMEM((C, DW), jnp.int32),
        pltpu.VMEM((C, G), jnp.float32),
        pltpu.VMEM((C,), jnp.int32),
        pltpu.VMEM((C * G,), jnp.float32),
        pltpu.VMEM((C * D,), jnp.float32),
        pltpu.SemaphoreType.DMA,
    ],
)(_body)


@jax.jit
def kernel(x, qvals, scales, zeros):
    xf = x.reshape(-1)
    q32 = lax.bitcast_convert_type(qvals.reshape(NUM_EMB, DW, 4), jnp.int32)
    z32 = lax.bitcast_convert_type(zeros, jnp.int16).astype(jnp.int32)
    out = _sc_call(xf, q32, scales, z32)
    return out.reshape(*x.shape, D)


# speedup vs baseline: 2.2544x; 1.1900x over previous
import functools

import jax
import jax.numpy as jnp
from jax import lax
from jax.experimental import pallas as pl
from jax.experimental.pallas import tpu as pltpu
from jax.experimental.pallas import tpu_sc as plsc

NUM_EMB = 1000000
D = 64
DW = D // 4
G = 2
T = 4096 * 50
NW = 32
N_PER = T // NW
C = 128
NCHUNK = N_PER // C


def _body(x_ref, q_ref, s_ref, z_ref, out_ref, idx_v, qv, sv, zv, zf, ov, sem):
    nc = 2
    wid = lax.axis_index("s") * nc + lax.axis_index("c")
    base = wid * N_PER

    lane = lax.iota(jnp.int32, 16)
    halfsel = (lane >= 8).astype(jnp.int32)
    e2 = lane * 2
    cols = [lane * 4 + k for k in range(4)]

    def chunk_body(ci, carry):
        cbase = base + ci * C
        pltpu.sync_copy(x_ref.at[pl.ds(cbase, C)], idx_v)
        cp_q = pltpu.async_copy(q_ref.at[idx_v], qv, sem)
        cp_s = pltpu.async_copy(s_ref.at[idx_v], sv, sem)
        cp_z = pltpu.async_copy(z_ref.at[idx_v], zv, sem)
        cp_q.wait()
        cp_s.wait()
        cp_z.wait()

        def pre_body(j, pcarry):
            zw = zv[pl.ds(j * 16, 16)]
            z0 = ((zw << 24) >> 24).astype(jnp.float32)
            z1 = ((zw << 16) >> 24).astype(jnp.float32)
            pbase = jnp.broadcast_to(j * 32, (16,))
            plsc.store_scatter(zf, [pbase + e2], z0)
            plsc.store_scatter(zf, [pbase + e2 + 1], z1)
            return pcarry

        lax.fori_loop(0, C // 16, pre_body, 0)

        def row_body(i, rcarry):
            w = plsc.bitcast(qv[i, :], jnp.int32)
            b0 = (w << 24) >> 24
            b1 = (w << 16) >> 24
            b2 = (w << 8) >> 24
            b3 = w >> 24
            rowv = jnp.broadcast_to(i, (16,))
            idx_sz = jnp.broadcast_to(i * 2, (16,)) + halfsel
            svec = plsc.load_gather(sv, [rowv, halfsel])
            zvec = plsc.load_gather(zf, [idx_sz])
            obase = jnp.broadcast_to(i * 64, (16,))
            for k, bk in enumerate((b0, b1, b2, b3)):
                fk = bk.astype(jnp.float32)
                plsc.store_scatter(ov, [obase + cols[k]], (fk - zvec) * svec)
            return rcarry

        lax.fori_loop(0, C, row_body, 0)
        pltpu.sync_copy(ov, out_ref.at[pl.ds(cbase * D, C * D)])
        return carry

    lax.fori_loop(0, NCHUNK, chunk_body, 0)


_sc_call = functools.partial(
    pl.kernel,
    out_type=jax.ShapeDtypeStruct((T * D,), jnp.float32),
    mesh=plsc.VectorSubcoreMesh(core_axis_name="c", subcore_axis_name="s"),
    compiler_params=pltpu.CompilerParams(
        needs_layout_passes=False, use_tc_tiling_on_sc=False),
    scratch_types=[
        pltpu.VMEM((C,), jnp.int32),
        pltpu.VMEM((C, D), jnp.int8),
        pltpu.VMEM((C, G), jnp.float32),
        pltpu.VMEM((C,), jnp.int32),
        pltpu.VMEM((C * G,), jnp.float32),
        pltpu.VMEM((C * D,), jnp.float32),
        pltpu.SemaphoreType.DMA,
    ],
)(_body)


@jax.jit
def kernel(x, qvals, scales, zeros):
    xf = x.reshape(-1)
    z32 = lax.bitcast_convert_type(zeros, jnp.int16).astype(jnp.int32)
    out = _sc_call(xf, qvals, scales, z32)
    return out.reshape(*x.shape, D)
